# Initial kernel scaffold; baseline (speedup 1.0000x reference)
import functools

import jax
import jax.numpy as jnp
from jax import lax
from jax.experimental import pallas as pl
from jax.experimental.pallas import tpu as pltpu
from jax.experimental.pallas import tpu_sc as plsc

_EPS = 1e-5
_K = 16
_ROWS = 256
_G = 512
_CH = 128


def _knn_call(points, ptsT):
    B, N, _ = points.shape

    def body(pts_ref, ptsT_ref, idx_ref):
        b = pl.program_id(0)
        pr = pts_ref[0]
        pt = ptsT_ref[0]
        sqr = jnp.sum(pr * pr, axis=1, keepdims=True)
        sqc = jnp.sum(pt * pt, axis=0, keepdims=True)
        d = sqr + sqc - 2.0 * jnp.dot(pr, pt, preferred_element_type=jnp.float32)
        iota = lax.broadcasted_iota(jnp.int32, d.shape, 1)
        base = b * N
        cols = []
        for _ in range(_K):
            m = jnp.min(d, axis=1, keepdims=True)
            ind = jnp.min(jnp.where(d == m, iota, N), axis=1, keepdims=True)
            cols.append(ind + base)
            d = jnp.where(iota == ind, jnp.inf, d)
        idx_ref[0] = jnp.concatenate(cols, axis=1)

    return pl.pallas_call(
        body,
        grid=(B, N // _ROWS),
        in_specs=[
            pl.BlockSpec((1, _ROWS, 3), lambda b, i: (b, i, 0)),
            pl.BlockSpec((1, 3, N), lambda b, i: (b, 0, 0)),
        ],
        out_specs=pl.BlockSpec((1, _ROWS, _K), lambda b, i: (b, i, 0)),
        out_shape=jax.ShapeDtypeStruct((B, N, _K), jnp.int32),
    )(points, ptsT)


def _sc_gather(tab, idxf):
    _, DW = tab.shape
    T = idxf.shape[0]
    info = plsc.get_sparse_core_info()
    nw = info.num_cores * info.num_subcores
    q = T // nw
    nch = q // _CH
    mesh = plsc.VectorSubcoreMesh(core_axis_name="c", subcore_axis_name="s")

    @functools.partial(
        pl.kernel, mesh=mesh,
        out_type=jax.ShapeDtypeStruct((T, DW), jnp.float32),
        scratch_types=[
            pltpu.VMEM((_CH,), jnp.int32),
            pltpu.VMEM((_CH, DW), jnp.float32),
            pltpu.SemaphoreType.DMA,
        ],
    )
    def k(tab_hbm, idx_hbm, out_hbm, idx_v, rows_v, sem):
        wid = lax.axis_index("s") * info.num_cores + lax.axis_index("c")
        base = wid * q

        def body(ci, carry):
            off = base + ci * _CH
            pltpu.sync_copy(idx_hbm.at[pl.ds(off, _CH)], idx_v)
            pltpu.async_copy(tab_hbm.at[idx_v], rows_v, sem).wait()
            pltpu.sync_copy(rows_v, out_hbm.at[pl.ds(off, _CH)])
            return carry

        lax.fori_loop(0, nch, body, 0)

    return k(tab, idxf)


def kernel(points, features, W_geom, g_geom, b_geom, W_sem, g_sem, b_sem,
           W_fuse, g_fuse, b_fuse):
    B, N, _ = points.shape
    C = features.shape[2]
    O = W_geom.shape[0]
    BN = B * N
    M = float(BN * _K)
    f32 = jnp.float32

    ptsT = jnp.transpose(points, (0, 2, 1))
    idx_g = _knn_call(points, ptsT)

    DW = 48
    tab = jnp.concatenate([features, points], axis=-1).reshape(BN, C + 3)
    tab = jnp.pad(tab, ((0, 0), (0, DW - (C + 3))))
    idxT = idx_g.reshape(BN, _K).T.reshape(_K * BN)
    gath = _sc_gather(tab, idxT).reshape(_K, BN, DW)

    cp = points.reshape(BN, 3)
    cf = features.reshape(BN, C)

    wg2 = W_geom[:, 3:6].T
    wg12 = (W_geom[:, 0:3] - W_geom[:, 3:6]).T
    ws2 = W_sem[:, C:2 * C].T
    ws12 = (W_sem[:, 0:C] - W_sem[:, C:2 * C]).T
    wf1 = W_fuse[:, 0:O].T
    wf2 = W_fuse[:, O:2 * O].T

    gath_spec = pl.BlockSpec((_K, _G, DW), lambda i: (0, i, 0))
    row_spec3 = pl.BlockSpec((_G, 3), lambda i: (i, 0))
    row_specC = pl.BlockSpec((_G, C), lambda i: (i, 0))
    full = lambda *shape: pl.BlockSpec(shape, lambda i: tuple(0 for _ in shape))
    nsteps = BN // _G

    def _y12(g_ref, wg2_ref, ws2_ref, j, t1, t2):
        gf = g_ref[j, :, 0:C]
        gp = g_ref[j, :, C:C + 3]
        y1 = jnp.dot(gp, wg2_ref[...], preferred_element_type=f32) + t1
        y2 = jnp.dot(gf, ws2_ref[...], preferred_element_type=f32) + t2
        return y1, y2

    def _t12(cp_ref, cf_ref, wg12_ref, ws12_ref):
        t1 = jnp.dot(cp_ref[...], wg12_ref[...], preferred_element_type=f32)
        t2 = jnp.dot(cf_ref[...], ws12_ref[...], preferred_element_type=f32)
        return t1, t2

    def p1(g_ref, cp_ref, cf_ref, wg2_ref, wg12_ref, ws2_ref, ws12_ref, mom_ref):
        t1, t2 = _t12(cp_ref, cf_ref, wg12_ref, ws12_ref)
        ay1 = jnp.zeros_like(t1)
        aq1 = jnp.zeros_like(t1)
        ay2 = jnp.zeros_like(t2)
        aq2 = jnp.zeros_like(t2)
        for j in range(_K):
            y1, y2 = _y12(g_ref, wg2_ref, ws2_ref, j, t1, t2)
            ay1 += y1
            aq1 += y1 * y1
            ay2 += y2
            aq2 += y2 * y2
        rows = jnp.concatenate(
            [jnp.sum(a, axis=0, keepdims=True) for a in (ay1, aq1, ay2, aq2)],
            axis=0)

        @pl.when(pl.program_id(0) == 0)
        def _():
            mom_ref[...] = jnp.zeros_like(mom_ref)

        mom_ref[0:4, :] = mom_ref[0:4, :] + rows

    mom = pl.pallas_call(
        p1,
        grid=(nsteps,),
        in_specs=[gath_spec, row_spec3, row_specC, full(3, O), full(3, O),
                  full(C, O), full(C, O)],
        out_specs=full(8, O),
        out_shape=jax.ShapeDtypeStruct((8, O), f32),
    )(gath, cp, cf, wg2, wg12, ws2, ws12)

    mean1 = mom[0] / M
    var1 = mom[1] / M - mean1 * mean1
    a1 = g_geom / jnp.sqrt(var1 + _EPS)
    c1 = b_geom - mean1 * a1
    mean2 = mom[2] / M
    var2 = mom[3] / M - mean2 * mean2
    a2 = g_sem / jnp.sqrt(var2 + _EPS)
    c2 = b_sem - mean2 * a2
    coef12 = jnp.zeros((8, O), f32).at[0].set(a1).at[1].set(c1).at[2].set(a2).at[3].set(c2)

    def p2(g_ref, cp_ref, cf_ref, wg2_ref, wg12_ref, ws2_ref, ws12_ref,
           wf1_ref, wf2_ref, coef_ref, mom_ref):
        t1, t2 = _t12(cp_ref, cf_ref, wg12_ref, ws12_ref)
        va1 = coef_ref[0:1, :]
        vc1 = coef_ref[1:2, :]
        va2 = coef_ref[2:3, :]
        vc2 = coef_ref[3:4, :]
        ay3 = jnp.zeros((t1.shape[0], O), f32)
        aq3 = jnp.zeros((t1.shape[0], O), f32)
        for j in range(_K):
            y1, y2 = _y12(g_ref, wg2_ref, ws2_ref, j, t1, t2)
            h1 = jnp.maximum(y1 * va1 + vc1, 0.0)
            h2 = jnp.maximum(y2 * va2 + vc2, 0.0)
            y3 = (jnp.dot(h1, wf1_ref[...], preferred_element_type=f32)
                  + jnp.dot(h2, wf2_ref[...], preferred_element_type=f32))
            ay3 += y3
            aq3 += y3 * y3
        rows = jnp.concatenate(
            [jnp.sum(a, axis=0, keepdims=True) for a in (ay3, aq3)], axis=0)

        @pl.when(pl.program_id(0) == 0)
        def _():
            mom_ref[...] = jnp.zeros_like(mom_ref)

        mom_ref[0:2, :] = mom_ref[0:2, :] + rows

    mom3 = pl.pallas_call(
        p2,
        grid=(nsteps,),
        in_specs=[gath_spec, row_spec3, row_specC, full(3, O), full(3, O),
                  full(C, O), full(C, O), full(O, O), full(O, O), full(8, O)],
        out_specs=full(8, O),
        out_shape=jax.ShapeDtypeStruct((8, O), f32),
    )(gath, cp, cf, wg2, wg12, ws2, ws12, wf1, wf2, coef12)

    mean3 = mom3[0] / M
    var3 = mom3[1] / M - mean3 * mean3
    a3 = g_fuse / jnp.sqrt(var3 + _EPS)
    c3 = b_fuse - mean3 * a3
    coef = coef12.at[4].set(a3).at[5].set(c3)

    def p3(g_ref, cp_ref, cf_ref, wg2_ref, wg12_ref, ws2_ref, ws12_ref,
           wf1_ref, wf2_ref, coef_ref, out_ref):
        t1, t2 = _t12(cp_ref, cf_ref, wg12_ref, ws12_ref)
        va1 = coef_ref[0:1, :]
        vc1 = coef_ref[1:2, :]
        va2 = coef_ref[2:3, :]
        vc2 = coef_ref[3:4, :]
        va3 = coef_ref[4:5, :]
        vc3 = coef_ref[5:6, :]
        acc = jnp.full((t1.shape[0], O), -jnp.inf, f32)
        for j in range(_K):
            y1, y2 = _y12(g_ref, wg2_ref, ws2_ref, j, t1, t2)
            h1 = jnp.maximum(y1 * va1 + vc1, 0.0)
            h2 = jnp.maximum(y2 * va2 + vc2, 0.0)
            y3 = (jnp.dot(h1, wf1_ref[...], preferred_element_type=f32)
                  + jnp.dot(h2, wf2_ref[...], preferred_element_type=f32))
            acc = jnp.maximum(acc, jnp.maximum(y3 * va3 + vc3, 0.0))
        out_ref[...] = acc

    out = pl.pallas_call(
        p3,
        grid=(nsteps,),
        in_specs=[gath_spec, row_spec3, row_specC, full(3, O), full(3, O),
                  full(C, O), full(C, O), full(O, O), full(O, O), full(8, O)],
        out_specs=pl.BlockSpec((_G, O), lambda i: (i, 0)),
        out_shape=jax.ShapeDtypeStruct((BN, O), f32),
    )(gath, cp, cf, wg2, wg12, ws2, ws12, wf1, wf2, coef)

    return out.reshape(B, N, O)


# speedup vs baseline: 14.8696x; 14.8696x over previous
import functools

import jax
import jax.numpy as jnp
from jax import lax
from jax.experimental import pallas as pl
from jax.experimental.pallas import tpu as pltpu
from jax.experimental.pallas import tpu_sc as plsc

_EPS = 1e-5
_K = 16
_ROWS = 256
_G = 512
_CH = 128


def _knn_call(points, ptsT):
    B, N, _ = points.shape

    def body(pts_ref, ptsT_ref, idx_ref):
        b = pl.program_id(0)
        pr = pts_ref[0]
        pt = ptsT_ref[0]
        sqr = jnp.sum(pr * pr, axis=1, keepdims=True)
        sqc = jnp.sum(pt * pt, axis=0, keepdims=True)
        d = sqr + sqc - 2.0 * jnp.dot(pr, pt, preferred_element_type=jnp.float32)
        iota = lax.broadcasted_iota(jnp.int32, d.shape, 1)
        base = b * N
        cols = []
        for _ in range(_K):
            m = jnp.min(d, axis=1, keepdims=True)
            ind = jnp.min(jnp.where(d == m, iota, N), axis=1, keepdims=True)
            cols.append(ind + base)
            d = jnp.where(iota == ind, jnp.inf, d)
        idx_ref[0] = jnp.concatenate(cols, axis=1)

    return pl.pallas_call(
        body,
        grid=(B, N // _ROWS),
        in_specs=[
            pl.BlockSpec((1, _ROWS, 3), lambda b, i: (b, i, 0)),
            pl.BlockSpec((1, 3, N), lambda b, i: (b, 0, 0)),
        ],
        out_specs=pl.BlockSpec((1, _ROWS, _K), lambda b, i: (b, i, 0)),
        out_shape=jax.ShapeDtypeStruct((B, N, _K), jnp.int32),
    )(points, ptsT)


def _sc_gather(tab, idxf):
    _, DW = tab.shape
    T = idxf.shape[0]
    info = plsc.get_sparse_core_info()
    nw = info.num_cores * info.num_subcores
    q = T // nw
    nch = q // _CH
    mesh = plsc.VectorSubcoreMesh(core_axis_name="c", subcore_axis_name="s")

    @functools.partial(
        pl.kernel, mesh=mesh,
        compiler_params=pltpu.CompilerParams(use_tc_tiling_on_sc=False),
        out_type=jax.ShapeDtypeStruct((T, DW), jnp.float32),
        scratch_types=[
            pltpu.VMEM((_CH,), jnp.int32),
            pltpu.VMEM((_CH, DW), jnp.float32),
            pltpu.SemaphoreType.DMA,
        ],
    )
    def k(tab_hbm, idx_hbm, out_hbm, idx_v, rows_v, sem):
        wid = lax.axis_index("s") * info.num_cores + lax.axis_index("c")
        base = wid * q

        def body(ci, carry):
            off = base + ci * _CH
            pltpu.sync_copy(idx_hbm.at[pl.ds(off, _CH)], idx_v)
            pltpu.async_copy(tab_hbm.at[idx_v], rows_v, sem).wait()
            pltpu.sync_copy(rows_v, out_hbm.at[pl.ds(off, _CH)])
            return carry

        lax.fori_loop(0, nch, body, 0)

    return k(tab, idxf)


def kernel(points, features, W_geom, g_geom, b_geom, W_sem, g_sem, b_sem,
           W_fuse, g_fuse, b_fuse):
    B, N, _ = points.shape
    C = features.shape[2]
    O = W_geom.shape[0]
    BN = B * N
    M = float(BN * _K)
    f32 = jnp.float32

    ptsT = jnp.transpose(points, (0, 2, 1))
    idx_g = _knn_call(points, ptsT)

    DW = 48
    tab = jnp.concatenate([features, points], axis=-1).reshape(BN, C + 3)
    tab = jnp.pad(tab, ((0, 0), (0, DW - (C + 3))))
    idxT = idx_g.reshape(BN, _K).T.reshape(_K * BN)
    gath = _sc_gather(tab, idxT).reshape(_K, BN, DW)

    cp = points.reshape(BN, 3)
    cf = features.reshape(BN, C)

    wg2 = W_geom[:, 3:6].T
    wg12 = (W_geom[:, 0:3] - W_geom[:, 3:6]).T
    ws2 = W_sem[:, C:2 * C].T
    ws12 = (W_sem[:, 0:C] - W_sem[:, C:2 * C]).T
    wf1 = W_fuse[:, 0:O].T
    wf2 = W_fuse[:, O:2 * O].T

    gath_spec = pl.BlockSpec((_K, _G, DW), lambda i: (0, i, 0))
    row_spec3 = pl.BlockSpec((_G, 3), lambda i: (i, 0))
    row_specC = pl.BlockSpec((_G, C), lambda i: (i, 0))
    full = lambda *shape: pl.BlockSpec(shape, lambda i: tuple(0 for _ in shape))
    nsteps = BN // _G

    def _y12(g_ref, wg2_ref, ws2_ref, j, t1, t2):
        gf = g_ref[j, :, 0:C]
        gp = g_ref[j, :, C:C + 3]
        y1 = jnp.dot(gp, wg2_ref[...], preferred_element_type=f32) + t1
        y2 = jnp.dot(gf, ws2_ref[...], preferred_element_type=f32) + t2
        return y1, y2

    def _t12(cp_ref, cf_ref, wg12_ref, ws12_ref):
        t1 = jnp.dot(cp_ref[...], wg12_ref[...], preferred_element_type=f32)
        t2 = jnp.dot(cf_ref[...], ws12_ref[...], preferred_element_type=f32)
        return t1, t2

    def p1(g_ref, cp_ref, cf_ref, wg2_ref, wg12_ref, ws2_ref, ws12_ref, mom_ref):
        t1, t2 = _t12(cp_ref, cf_ref, wg12_ref, ws12_ref)
        ay1 = jnp.zeros_like(t1)
        aq1 = jnp.zeros_like(t1)
        ay2 = jnp.zeros_like(t2)
        aq2 = jnp.zeros_like(t2)
        for j in range(_K):
            y1, y2 = _y12(g_ref, wg2_ref, ws2_ref, j, t1, t2)
            ay1 += y1
            aq1 += y1 * y1
            ay2 += y2
            aq2 += y2 * y2
        rows = jnp.concatenate(
            [jnp.sum(a, axis=0, keepdims=True) for a in (ay1, aq1, ay2, aq2)],
            axis=0)

        @pl.when(pl.program_id(0) == 0)
        def _():
            mom_ref[...] = jnp.zeros_like(mom_ref)

        mom_ref[0:4, :] = mom_ref[0:4, :] + rows

    mom = pl.pallas_call(
        p1,
        grid=(nsteps,),
        in_specs=[gath_spec, row_spec3, row_specC, full(3, O), full(3, O),
                  full(C, O), full(C, O)],
        out_specs=full(8, O),
        out_shape=jax.ShapeDtypeStruct((8, O), f32),
    )(gath, cp, cf, wg2, wg12, ws2, ws12)

    mean1 = mom[0] / M
    var1 = mom[1] / M - mean1 * mean1
    a1 = g_geom / jnp.sqrt(var1 + _EPS)
    c1 = b_geom - mean1 * a1
    mean2 = mom[2] / M
    var2 = mom[3] / M - mean2 * mean2
    a2 = g_sem / jnp.sqrt(var2 + _EPS)
    c2 = b_sem - mean2 * a2
    coef12 = jnp.zeros((8, O), f32).at[0].set(a1).at[1].set(c1).at[2].set(a2).at[3].set(c2)

    def p2(g_ref, cp_ref, cf_ref, wg2_ref, wg12_ref, ws2_ref, ws12_ref,
           wf1_ref, wf2_ref, coef_ref, mom_ref):
        t1, t2 = _t12(cp_ref, cf_ref, wg12_ref, ws12_ref)
        va1 = coef_ref[0:1, :]
        vc1 = coef_ref[1:2, :]
        va2 = coef_ref[2:3, :]
        vc2 = coef_ref[3:4, :]
        ay3 = jnp.zeros((t1.shape[0], O), f32)
        aq3 = jnp.zeros((t1.shape[0], O), f32)
        for j in range(_K):
            y1, y2 = _y12(g_ref, wg2_ref, ws2_ref, j, t1, t2)
            h1 = jnp.maximum(y1 * va1 + vc1, 0.0)
            h2 = jnp.maximum(y2 * va2 + vc2, 0.0)
            y3 = (jnp.dot(h1, wf1_ref[...], preferred_element_type=f32)
                  + jnp.dot(h2, wf2_ref[...], preferred_element_type=f32))
            ay3 += y3
            aq3 += y3 * y3
        rows = jnp.concatenate(
            [jnp.sum(a, axis=0, keepdims=True) for a in (ay3, aq3)], axis=0)

        @pl.when(pl.program_id(0) == 0)
        def _():
            mom_ref[...] = jnp.zeros_like(mom_ref)

        mom_ref[0:2, :] = mom_ref[0:2, :] + rows

    mom3 = pl.pallas_call(
        p2,
        grid=(nsteps,),
        in_specs=[gath_spec, row_spec3, row_specC, full(3, O), full(3, O),
                  full(C, O), full(C, O), full(O, O), full(O, O), full(8, O)],
        out_specs=full(8, O),
        out_shape=jax.ShapeDtypeStruct((8, O), f32),
    )(gath, cp, cf, wg2, wg12, ws2, ws12, wf1, wf2, coef12)

    mean3 = mom3[0] / M
    var3 = mom3[1] / M - mean3 * mean3
    a3 = g_fuse / jnp.sqrt(var3 + _EPS)
    c3 = b_fuse - mean3 * a3
    coef = coef12.at[4].set(a3).at[5].set(c3)

    def p3(g_ref, cp_ref, cf_ref, wg2_ref, wg12_ref, ws2_ref, ws12_ref,
           wf1_ref, wf2_ref, coef_ref, out_ref):
        t1, t2 = _t12(cp_ref, cf_ref, wg12_ref, ws12_ref)
        va1 = coef_ref[0:1, :]
        vc1 = coef_ref[1:2, :]
        va2 = coef_ref[2:3, :]
        vc2 = coef_ref[3:4, :]
        va3 = coef_ref[4:5, :]
        vc3 = coef_ref[5:6, :]
        acc = jnp.full((t1.shape[0], O), -jnp.inf, f32)
        for j in range(_K):
            y1, y2 = _y12(g_ref, wg2_ref, ws2_ref, j, t1, t2)
            h1 = jnp.maximum(y1 * va1 + vc1, 0.0)
            h2 = jnp.maximum(y2 * va2 + vc2, 0.0)
            y3 = (jnp.dot(h1, wf1_ref[...], preferred_element_type=f32)
                  + jnp.dot(h2, wf2_ref[...], preferred_element_type=f32))
            acc = jnp.maximum(acc, jnp.maximum(y3 * va3 + vc3, 0.0))
        out_ref[...] = acc

    out = pl.pallas_call(
        p3,
        grid=(nsteps,),
        in_specs=[gath_spec, row_spec3, row_specC, full(3, O), full(3, O),
                  full(C, O), full(C, O), full(O, O), full(O, O), full(8, O)],
        out_specs=pl.BlockSpec((_G, O), lambda i: (i, 0)),
        out_shape=jax.ShapeDtypeStruct((BN, O), f32),
    )(gath, cp, cf, wg2, wg12, ws2, ws12, wf1, wf2, coef)

    return out.reshape(B, N, O)


# speedup vs baseline: 15.0027x; 1.0089x over previous
import functools

import jax
import jax.numpy as jnp
from jax import lax
from jax.experimental import pallas as pl
from jax.experimental.pallas import tpu as pltpu
from jax.experimental.pallas import tpu_sc as plsc

_EPS = 1e-5
_K = 16
_ROWS = 256
_G = 512
_CH = 128


def _knn_call(points, ptsT):
    B, N, _ = points.shape

    def body(pts_ref, ptsT_ref, idx_ref):
        b = pl.program_id(0)
        pr = pts_ref[0]
        pt = ptsT_ref[0]
        sqr = jnp.sum(pr * pr, axis=1, keepdims=True)
        sqc = jnp.sum(pt * pt, axis=0, keepdims=True)
        d = sqr + sqc - 2.0 * jnp.dot(pr, pt, preferred_element_type=jnp.float32)
        iota = lax.broadcasted_iota(jnp.int32, d.shape, 1)
        i32 = jnp.int32
        imax = i32(0x7FFFFFFF)
        hmask = i32(-4096)

        def skey(x):
            bits = lax.bitcast_convert_type(x, i32)
            return jnp.where(bits < 0, bits ^ imax, bits)

        key = (skey(d) & hmask) | iota
        ms = []
        for _ in range(_K):
            m = jnp.min(key, axis=1, keepdims=True)
            ms.append(m)
            key = jnp.where(key == m, imax, key)
        mstack = jnp.concatenate(ms, axis=1)
        dpart = mstack & hmask
        idx_sel = mstack & i32(0xFFF)

        q = dpart[:, _K - 1:_K]
        clsn = dpart == q
        r_row = jnp.sum(clsn.astype(i32), axis=1, keepdims=True)
        ranks = []
        acc = jnp.zeros_like(r_row)
        for t in range(_K):
            ranks.append(acc)
            acc = acc + clsn[:, t:t + 1].astype(i32)
        rank = jnp.concatenate(ranks, axis=1)
        dc = jnp.where((skey(d) & hmask) == q, d, jnp.inf)
        maxr = jnp.max(r_row)

        def fix_body(carry):
            i, dcur, oidx = carry
            md = jnp.min(dcur, axis=1, keepdims=True)
            mind = jnp.min(jnp.where(dcur == md, iota, N), axis=1, keepdims=True)
            dcur = jnp.where(iota == mind, jnp.inf, dcur)
            oidx = jnp.where(clsn & (rank == i) & (r_row > i), mind, oidx)
            return i + 1, dcur, oidx

        _, _, idx_fix = lax.while_loop(lambda c: c[0] < maxr, fix_body,
                                       (i32(0), dc, idx_sel))
        idx_ref[0] = idx_fix + b * N

    return pl.pallas_call(
        body,
        grid=(B, N // _ROWS),
        in_specs=[
            pl.BlockSpec((1, _ROWS, 3), lambda b, i: (b, i, 0)),
            pl.BlockSpec((1, 3, N), lambda b, i: (b, 0, 0)),
        ],
        out_specs=pl.BlockSpec((1, _ROWS, _K), lambda b, i: (b, i, 0)),
        out_shape=jax.ShapeDtypeStruct((B, N, _K), jnp.int32),
    )(points, ptsT)


def _sc_gather(tab, idxf):
    _, DW = tab.shape
    T = idxf.shape[0]
    info = plsc.get_sparse_core_info()
    nw = info.num_cores * info.num_subcores
    q = T // nw
    nch = q // _CH
    mesh = plsc.VectorSubcoreMesh(core_axis_name="c", subcore_axis_name="s")

    @functools.partial(
        pl.kernel, mesh=mesh,
        compiler_params=pltpu.CompilerParams(use_tc_tiling_on_sc=False),
        out_type=jax.ShapeDtypeStruct((T, DW), jnp.float32),
        scratch_types=[
            pltpu.VMEM((_CH,), jnp.int32),
            pltpu.VMEM((_CH, DW), jnp.float32),
            pltpu.SemaphoreType.DMA,
        ],
    )
    def k(tab_hbm, idx_hbm, out_hbm, idx_v, rows_v, sem):
        wid = lax.axis_index("s") * info.num_cores + lax.axis_index("c")
        base = wid * q

        def body(ci, carry):
            off = base + ci * _CH
            pltpu.sync_copy(idx_hbm.at[pl.ds(off, _CH)], idx_v)
            pltpu.async_copy(tab_hbm.at[idx_v], rows_v, sem).wait()
            pltpu.sync_copy(rows_v, out_hbm.at[pl.ds(off, _CH)])
            return carry

        lax.fori_loop(0, nch, body, 0)

    return k(tab, idxf)


def kernel(points, features, W_geom, g_geom, b_geom, W_sem, g_sem, b_sem,
           W_fuse, g_fuse, b_fuse):
    B, N, _ = points.shape
    C = features.shape[2]
    O = W_geom.shape[0]
    BN = B * N
    M = float(BN * _K)
    f32 = jnp.float32

    ptsT = jnp.transpose(points, (0, 2, 1))
    idx_g = _knn_call(points, ptsT)

    DW = 48
    tab = jnp.concatenate([features, points], axis=-1).reshape(BN, C + 3)
    tab = jnp.pad(tab, ((0, 0), (0, DW - (C + 3))))
    idxT = idx_g.reshape(BN, _K).T.reshape(_K * BN)
    gath = _sc_gather(tab, idxT).reshape(_K, BN, DW)

    cp = points.reshape(BN, 3)
    cf = features.reshape(BN, C)

    wg2 = W_geom[:, 3:6].T
    wg12 = (W_geom[:, 0:3] - W_geom[:, 3:6]).T
    ws2 = W_sem[:, C:2 * C].T
    ws12 = (W_sem[:, 0:C] - W_sem[:, C:2 * C]).T
    wf1 = W_fuse[:, 0:O].T
    wf2 = W_fuse[:, O:2 * O].T

    gath_spec = pl.BlockSpec((_K, _G, DW), lambda i: (0, i, 0))
    row_spec3 = pl.BlockSpec((_G, 3), lambda i: (i, 0))
    row_specC = pl.BlockSpec((_G, C), lambda i: (i, 0))
    full = lambda *shape: pl.BlockSpec(shape, lambda i: tuple(0 for _ in shape))
    nsteps = BN // _G

    def _y12(g_ref, wg2_ref, ws2_ref, j, t1, t2):
        gf = g_ref[j, :, 0:C]
        gp = g_ref[j, :, C:C + 3]
        y1 = jnp.dot(gp, wg2_ref[...], preferred_element_type=f32) + t1
        y2 = jnp.dot(gf, ws2_ref[...], preferred_element_type=f32) + t2
        return y1, y2

    def _t12(cp_ref, cf_ref, wg12_ref, ws12_ref):
        t1 = jnp.dot(cp_ref[...], wg12_ref[...], preferred_element_type=f32)
        t2 = jnp.dot(cf_ref[...], ws12_ref[...], preferred_element_type=f32)
        return t1, t2

    def p1(g_ref, cp_ref, cf_ref, wg2_ref, wg12_ref, ws2_ref, ws12_ref, mom_ref):
        t1, t2 = _t12(cp_ref, cf_ref, wg12_ref, ws12_ref)
        ay1 = jnp.zeros_like(t1)
        aq1 = jnp.zeros_like(t1)
        ay2 = jnp.zeros_like(t2)
        aq2 = jnp.zeros_like(t2)
        for j in range(_K):
            y1, y2 = _y12(g_ref, wg2_ref, ws2_ref, j, t1, t2)
            ay1 += y1
            aq1 += y1 * y1
            ay2 += y2
            aq2 += y2 * y2
        rows = jnp.concatenate(
            [jnp.sum(a, axis=0, keepdims=True) for a in (ay1, aq1, ay2, aq2)],
            axis=0)

        @pl.when(pl.program_id(0) == 0)
        def _():
            mom_ref[...] = jnp.zeros_like(mom_ref)

        mom_ref[0:4, :] = mom_ref[0:4, :] + rows

    mom = pl.pallas_call(
        p1,
        grid=(nsteps,),
        in_specs=[gath_spec, row_spec3, row_specC, full(3, O), full(3, O),
                  full(C, O), full(C, O)],
        out_specs=full(8, O),
        out_shape=jax.ShapeDtypeStruct((8, O), f32),
    )(gath, cp, cf, wg2, wg12, ws2, ws12)

    mean1 = mom[0] / M
    var1 = mom[1] / M - mean1 * mean1
    a1 = g_geom / jnp.sqrt(var1 + _EPS)
    c1 = b_geom - mean1 * a1
    mean2 = mom[2] / M
    var2 = mom[3] / M - mean2 * mean2
    a2 = g_sem / jnp.sqrt(var2 + _EPS)
    c2 = b_sem - mean2 * a2
    coef12 = jnp.zeros((8, O), f32).at[0].set(a1).at[1].set(c1).at[2].set(a2).at[3].set(c2)

    def p2(g_ref, cp_ref, cf_ref, wg2_ref, wg12_ref, ws2_ref, ws12_ref,
           wf1_ref, wf2_ref, coef_ref, mom_ref):
        t1, t2 = _t12(cp_ref, cf_ref, wg12_ref, ws12_ref)
        va1 = coef_ref[0:1, :]
        vc1 = coef_ref[1:2, :]
        va2 = coef_ref[2:3, :]
        vc2 = coef_ref[3:4, :]
        ay3 = jnp.zeros((t1.shape[0], O), f32)
        aq3 = jnp.zeros((t1.shape[0], O), f32)
        for j in range(_K):
            y1, y2 = _y12(g_ref, wg2_ref, ws2_ref, j, t1, t2)
            h1 = jnp.maximum(y1 * va1 + vc1, 0.0)
            h2 = jnp.maximum(y2 * va2 + vc2, 0.0)
            y3 = (jnp.dot(h1, wf1_ref[...], preferred_element_type=f32)
                  + jnp.dot(h2, wf2_ref[...], preferred_element_type=f32))
            ay3 += y3
            aq3 += y3 * y3
        rows = jnp.concatenate(
            [jnp.sum(a, axis=0, keepdims=True) for a in (ay3, aq3)], axis=0)

        @pl.when(pl.program_id(0) == 0)
        def _():
            mom_ref[...] = jnp.zeros_like(mom_ref)

        mom_ref[0:2, :] = mom_ref[0:2, :] + rows

    mom3 = pl.pallas_call(
        p2,
        grid=(nsteps,),
        in_specs=[gath_spec, row_spec3, row_specC, full(3, O), full(3, O),
                  full(C, O), full(C, O), full(O, O), full(O, O), full(8, O)],
        out_specs=full(8, O),
        out_shape=jax.ShapeDtypeStruct((8, O), f32),
    )(gath, cp, cf, wg2, wg12, ws2, ws12, wf1, wf2, coef12)

    mean3 = mom3[0] / M
    var3 = mom3[1] / M - mean3 * mean3
    a3 = g_fuse / jnp.sqrt(var3 + _EPS)
    c3 = b_fuse - mean3 * a3
    coef = coef12.at[4].set(a3).at[5].set(c3)

    def p3(g_ref, cp_ref, cf_ref, wg2_ref, wg12_ref, ws2_ref, ws12_ref,
           wf1_ref, wf2_ref, coef_ref, out_ref):
        t1, t2 = _t12(cp_ref, cf_ref, wg12_ref, ws12_ref)
        va1 = coef_ref[0:1, :]
        vc1 = coef_ref[1:2, :]
        va2 = coef_ref[2:3, :]
        vc2 = coef_ref[3:4, :]
        va3 = coef_ref[4:5, :]
        vc3 = coef_ref[5:6, :]
        acc = jnp.full((t1.shape[0], O), -jnp.inf, f32)
        for j in range(_K):
            y1, y2 = _y12(g_ref, wg2_ref, ws2_ref, j, t1, t2)
            h1 = jnp.maximum(y1 * va1 + vc1, 0.0)
            h2 = jnp.maximum(y2 * va2 + vc2, 0.0)
            y3 = (jnp.dot(h1, wf1_ref[...], preferred_element_type=f32)
                  + jnp.dot(h2, wf2_ref[...], preferred_element_type=f32))
            acc = jnp.maximum(acc, jnp.maximum(y3 * va3 + vc3, 0.0))
        out_ref[...] = acc

    out = pl.pallas_call(
        p3,
        grid=(nsteps,),
        in_specs=[gath_spec, row_spec3, row_specC, full(3, O), full(3, O),
                  full(C, O), full(C, O), full(O, O), full(O, O), full(8, O)],
        out_specs=pl.BlockSpec((_G, O), lambda i: (i, 0)),
        out_shape=jax.ShapeDtypeStruct((BN, O), f32),
    )(gath, cp, cf, wg2, wg12, ws2, ws12, wf1, wf2, coef)

    return out.reshape(B, N, O)
